# baseline (speedup 1.0000x reference)
import functools

import jax
import jax.numpy as jnp
from jax import lax
from jax.experimental import pallas as pl
from jax.experimental.pallas import tpu as pltpu
from jax.experimental.pallas import tpu_sc as plsc

N = 10000
E = 320000
DF = 128
DO = 64
NC = 2
NS = 16
EPT = E // (NC * NS)
CH = 80
NCHUNK = EPT // CH
NBUF = 5
ABUF = 10
SLACK = 5
RPT = N // NS
DEGW = 16

_MESH = plsc.VectorSubcoreMesh(core_axis_name="c", subcore_axis_name="s")


def _copy_out(shared, out_hbm, c, s):
    pltpu.sync_copy(shared.at[pl.ds(s * 624, 624)],
                    out_hbm.at[c, pl.ds(s * 624, 624)])

    @pl.when(s == NS - 1)
    def _tail():
        pltpu.sync_copy(shared.at[pl.ds(9984, 16)],
                        out_hbm.at[c, pl.ds(9984, 16)])


def _deg_body(dst_hbm, out_hbm, idx_v, ones_v, zrow_v, deg_sh, sem):
    c = lax.axis_index("c")
    s = lax.axis_index("s")

    def fill(i, _):
        zrow_v[i, :] = jnp.zeros((DEGW,), jnp.float32)
        return _

    lax.fori_loop(0, 125, fill, None)

    def fill_o(i, _):
        ones_v[i, :] = jnp.ones((DEGW,), jnp.float32)
        return _

    lax.fori_loop(0, CH, fill_o, None)
    for t in range(5):
        pltpu.sync_copy(zrow_v, deg_sh.at[pl.ds(s * RPT + t * 125, 125)])
    pltpu.sync_copy(dst_hbm.at[c, s], idx_v)
    plsc.subcore_barrier()

    def grp(g, _):
        for b in range(NBUF):
            pltpu.async_copy(ones_v, deg_sh.at[idx_v.at[g * NBUF + b]], sem,
                             add=True)
        for b in range(NBUF):
            pltpu.make_async_copy(ones_v, deg_sh.at[idx_v.at[g * NBUF]],
                                  sem).wait()
        return _

    lax.fori_loop(0, NCHUNK // NBUF, grp, None)
    plsc.subcore_barrier()
    _copy_out(deg_sh, out_hbm, c, s)


_deg_call = functools.partial(
    pl.kernel,
    out_type=jax.ShapeDtypeStruct((NC, N, DEGW), jnp.float32),
    mesh=_MESH,
    scratch_types=[
        pltpu.VMEM((NCHUNK, CH), jnp.int32),
        pltpu.VMEM((CH, DEGW), jnp.float32),
        pltpu.VMEM((125, DEGW), jnp.float32),
        pltpu.VMEM_SHARED((N, DEGW), jnp.float32),
        pltpu.SemaphoreType.DMA,
    ],
)(_deg_body)


def _agg_body(g_hbm, src_hbm, dst_hbm, out_hbm, srci, dsti, rbs, zb, acc_sh,
              gsems, ssems):
    c = lax.axis_index("c")
    s = lax.axis_index("s")

    def fz(i, _):
        for k in range(DO // 16):
            zb[i, pl.ds(k * 16, 16)] = jnp.zeros((16,), jnp.float32)
        return _

    lax.fori_loop(0, 125, fz, None)
    for t in range(5):
        pltpu.sync_copy(zb, acc_sh.at[pl.ds(s * RPT + t * 125, 125)])
    pltpu.sync_copy(src_hbm.at[c, s], srci)
    pltpu.sync_copy(dst_hbm.at[c, s], dsti)
    plsc.subcore_barrier()

    def fire_gather(j, b):
        pltpu.async_copy(g_hbm.at[srci.at[j]], rbs[b], gsems[b])

    def wait_gather(j, b):
        pltpu.make_async_copy(g_hbm.at[srci.at[j]], rbs[b], gsems[b]).wait()

    def fire_scatter(j, b):
        pltpu.async_copy(rbs[b], acc_sh.at[dsti.at[j]], ssems[b], add=True)

    def wait_scatter(j, b):
        pltpu.make_async_copy(rbs[b], acc_sh.at[dsti.at[j]], ssems[b]).wait()

    for b in range(SLACK):
        fire_gather(b, b)

    def grp(g, _):
        for b in range(ABUF):
            j = g * ABUF + b
            wait_gather(j, b)
            fire_scatter(j, b)
            b2 = (b + SLACK) % ABUF
            if b < SLACK:
                pl.when(g > 0)(functools.partial(wait_scatter, j - SLACK, b2))
            else:
                wait_scatter(j - SLACK, b2)
            fire_gather(j + SLACK, b2)
        return _

    lax.fori_loop(0, (NCHUNK - SLACK) // ABUF, grp, None)
    for b2 in range(SLACK):
        j = NCHUNK - SLACK + b2
        b = j % ABUF
        wait_gather(j, b)
        fire_scatter(j, b)
    for j in range(NCHUNK - ABUF, NCHUNK):
        wait_scatter(j, j % ABUF)
    plsc.subcore_barrier()
    _copy_out(acc_sh, out_hbm, c, s)


_agg_call = functools.partial(
    pl.kernel,
    out_type=jax.ShapeDtypeStruct((NC, N, DO), jnp.float32),
    mesh=_MESH,
    scratch_types=[
        pltpu.VMEM((NCHUNK, CH), jnp.int32),
        pltpu.VMEM((NCHUNK, CH), jnp.int32),
        [pltpu.VMEM((CH, DO), jnp.float32)] * ABUF,
        pltpu.VMEM((125, DO), jnp.float32),
        pltpu.VMEM_SHARED((N, DO), jnp.float32),
        [pltpu.SemaphoreType.DMA] * ABUF,
        [pltpu.SemaphoreType.DMA] * ABUF,
    ],
    compiler_params=pltpu.CompilerParams(use_tc_tiling_on_sc=False),
)(_agg_body)


_BM = 1000


def _dinv_of(deg_ref):
    d = deg_ref[0] + deg_ref[1]
    return lax.rsqrt(1.0 + jnp.sum(d, axis=1, keepdims=True))


def _mm1_body(deg_ref, x_ref, w_ref, g_ref):
    dinv = _dinv_of(deg_ref)
    g_ref[...] = jnp.dot(x_ref[...], w_ref[...],
                         preferred_element_type=jnp.float32) * dinv


def _mm2_body(deg_ref, acc_ref, g0_ref, b0_ref, w1_ref, g1_ref):
    dinv = _dinv_of(deg_ref)
    t = (acc_ref[0] + acc_ref[1] + g0_ref[...]) * dinv + b0_ref[...]
    t = jnp.maximum(t, 0.0)
    g1_ref[...] = jnp.dot(t, w1_ref[...],
                          preferred_element_type=jnp.float32) * dinv


def _mm3_body(deg_ref, acc_ref, g1_ref, b1_ref, out_ref):
    dinv = _dinv_of(deg_ref)
    out_ref[...] = (acc_ref[0] + acc_ref[1] + g1_ref[...]) * dinv + b1_ref[...]


_deg_spec = pl.BlockSpec((NC, _BM, DEGW), lambda i: (0, i, 0))
_acc_spec = pl.BlockSpec((NC, _BM, DO), lambda i: (0, i, 0))
_row_spec = pl.BlockSpec((_BM, DO), lambda i: (i, 0))
_bias_spec = pl.BlockSpec((1, DO), lambda i: (0, 0))

_mm1 = pl.pallas_call(
    _mm1_body,
    grid=(N // _BM,),
    in_specs=[_deg_spec,
              pl.BlockSpec((_BM, DF), lambda i: (i, 0)),
              pl.BlockSpec((DF, DO), lambda i: (0, 0))],
    out_specs=_row_spec,
    out_shape=jax.ShapeDtypeStruct((N, DO), jnp.float32),
)

_mm2 = pl.pallas_call(
    _mm2_body,
    grid=(N // _BM,),
    in_specs=[_deg_spec, _acc_spec, _row_spec, _bias_spec,
              pl.BlockSpec((DO, DO), lambda i: (0, 0))],
    out_specs=_row_spec,
    out_shape=jax.ShapeDtypeStruct((N, DO), jnp.float32),
)

_mm3 = pl.pallas_call(
    _mm3_body,
    grid=(N // _BM,),
    in_specs=[_deg_spec, _acc_spec, _row_spec, _bias_spec],
    out_specs=_row_spec,
    out_shape=jax.ShapeDtypeStruct((N, DO), jnp.float32),
)


def kernel(x, edge_index, W0, b0, W1, b1):
    ei = edge_index.astype(jnp.int32)
    src_r = ei[0].reshape(NC, NS, NCHUNK, CH)
    dst_r = ei[1].reshape(NC, NS, NCHUNK, CH)
    deg_parts = _deg_call(dst_r)
    g0 = _mm1(deg_parts, x, W0)
    acc0 = _agg_call(g0, src_r, dst_r)
    g1 = _mm2(deg_parts, acc0, g0, b0.reshape(1, DO), W1)
    acc1 = _agg_call(g1, src_r, dst_r)
    out = _mm3(deg_parts, acc1, g1, b1.reshape(1, DO))
    return out


# speedup vs baseline: 1.2191x; 1.2191x over previous
import functools

import jax
import jax.numpy as jnp
from jax import lax
from jax.experimental import pallas as pl
from jax.experimental.pallas import tpu as pltpu
from jax.experimental.pallas import tpu_sc as plsc

N = 10000
E = 320000
DF = 128
DO = 64
NC = 2
NS = 16
EPT = E // (NC * NS)
CH = 80
NCHUNK = EPT // CH
NBUF = 5
ABUF = 10
SLACK = 5
RPT = N // NS
DEGW = 16

_MESH = plsc.VectorSubcoreMesh(core_axis_name="c", subcore_axis_name="s")


def _copy_out(shared, out_hbm, c, s):
    pltpu.sync_copy(shared.at[pl.ds(s * 624, 624)],
                    out_hbm.at[c, pl.ds(s * 624, 624)])

    @pl.when(s == NS - 1)
    def _tail():
        pltpu.sync_copy(shared.at[pl.ds(9984, 16)],
                        out_hbm.at[c, pl.ds(9984, 16)])


def _deg_body(dst_hbm, out_hbm, idx_v, ones_v, zrow_v, deg_sh, sem):
    c = lax.axis_index("c")
    s = lax.axis_index("s")

    def fill(i, _):
        zrow_v[i, :] = jnp.zeros((DEGW,), jnp.float32)
        return _

    lax.fori_loop(0, 125, fill, None)

    def fill_o(i, _):
        ones_v[i, :] = jnp.ones((DEGW,), jnp.float32)
        return _

    lax.fori_loop(0, CH, fill_o, None)
    for t in range(5):
        pltpu.sync_copy(zrow_v, deg_sh.at[pl.ds(s * RPT + t * 125, 125)])
    pltpu.sync_copy(dst_hbm.at[c, s], idx_v)
    plsc.subcore_barrier()

    def grp(g, _):
        for b in range(NBUF):
            pltpu.async_copy(ones_v, deg_sh.at[idx_v.at[g * NBUF + b]], sem,
                             add=True)
        for b in range(NBUF):
            pltpu.make_async_copy(ones_v, deg_sh.at[idx_v.at[g * NBUF]],
                                  sem).wait()
        return _

    lax.fori_loop(0, NCHUNK // NBUF, grp, None)
    plsc.subcore_barrier()
    _copy_out(deg_sh, out_hbm, c, s)


_deg_call = functools.partial(
    pl.kernel,
    out_type=jax.ShapeDtypeStruct((NC, N, DEGW), jnp.float32),
    mesh=_MESH,
    scratch_types=[
        pltpu.VMEM((NCHUNK, CH), jnp.int32),
        pltpu.VMEM((CH, DEGW), jnp.float32),
        pltpu.VMEM((125, DEGW), jnp.float32),
        pltpu.VMEM_SHARED((N, DEGW), jnp.float32),
        pltpu.SemaphoreType.DMA,
    ],
)(_deg_body)


def _agg_body(g_hbm, src_hbm, dst_hbm, out_hbm, srci, dsti, rbs, zb, acc_sh,
              gsems):
    c = lax.axis_index("c")
    s = lax.axis_index("s")

    def fz(i, _):
        for k in range(DO // 16):
            zb[i, pl.ds(k * 16, 16)] = jnp.zeros((16,), jnp.float32)
        return _

    lax.fori_loop(0, 125, fz, None)
    for t in range(5):
        pltpu.sync_copy(zb, acc_sh.at[pl.ds(s * RPT + t * 125, 125)])
    pltpu.sync_copy(src_hbm.at[c, s], srci)
    pltpu.sync_copy(dst_hbm.at[c, s], dsti)
    plsc.subcore_barrier()

    for b in range(NBUF):
        pltpu.async_copy(g_hbm.at[srci.at[b]], rbs[b], gsems[b])

    def grp(g, _):
        for b in range(NBUF):
            j = g * NBUF + b
            pltpu.make_async_copy(g_hbm.at[srci.at[j]], rbs[b],
                                  gsems[b]).wait()
            pltpu.sync_copy(rbs[b], acc_sh.at[dsti.at[j]], add=True)
            pltpu.async_copy(g_hbm.at[srci.at[j + NBUF]], rbs[b], gsems[b])
        return _

    lax.fori_loop(0, NCHUNK // NBUF - 1, grp, None)
    for b in range(NBUF):
        j = NCHUNK - NBUF + b
        pltpu.make_async_copy(g_hbm.at[srci.at[j]], rbs[b], gsems[b]).wait()
        pltpu.sync_copy(rbs[b], acc_sh.at[dsti.at[j]], add=True)
    plsc.subcore_barrier()
    _copy_out(acc_sh, out_hbm, c, s)


_agg_call = functools.partial(
    pl.kernel,
    out_type=jax.ShapeDtypeStruct((NC, N, DO), jnp.float32),
    mesh=_MESH,
    scratch_types=[
        pltpu.VMEM((NCHUNK, CH), jnp.int32),
        pltpu.VMEM((NCHUNK, CH), jnp.int32),
        [pltpu.VMEM((CH, DO), jnp.float32)] * NBUF,
        pltpu.VMEM((125, DO), jnp.float32),
        pltpu.VMEM_SHARED((N, DO), jnp.float32),
        [pltpu.SemaphoreType.DMA] * NBUF,
    ],
    compiler_params=pltpu.CompilerParams(use_tc_tiling_on_sc=False),
)(_agg_body)


NP = N // 2
_BM = 1000


def _dinvp_of(deg_ref):
    s = deg_ref[0] + deg_ref[1]
    de = lax.rsqrt(1.0 + jnp.sum(s[:, :DEGW], axis=1, keepdims=True))
    do = lax.rsqrt(1.0 + jnp.sum(s[:, DEGW:], axis=1, keepdims=True))
    return jnp.concatenate([jnp.broadcast_to(de, (_BM, DO)),
                            jnp.broadcast_to(do, (_BM, DO))], axis=1)


def _mm1_body(deg_ref, x_ref, w_ref, g_ref, dinvp_ref):
    dinvp = _dinvp_of(deg_ref)
    dinvp_ref[...] = dinvp
    g_ref[...] = jnp.dot(x_ref[...], w_ref[...],
                         preferred_element_type=jnp.float32) * dinvp


def _mm2_body(dinvp_ref, acc_ref, g0_ref, b0_ref, w1_ref, g1_ref):
    dinvp = dinvp_ref[...]
    t = (acc_ref[0] + acc_ref[1] + g0_ref[...]) * dinvp + b0_ref[...]
    t = jnp.maximum(t, 0.0)
    g1_ref[...] = jnp.dot(t, w1_ref[...],
                          preferred_element_type=jnp.float32) * dinvp


def _mm3_body(dinvp_ref, acc_ref, g1_ref, b1_ref, out_ref):
    out_ref[...] = ((acc_ref[0] + acc_ref[1] + g1_ref[...]) * dinvp_ref[...]
                    + b1_ref[...])


_deg_spec = pl.BlockSpec((NC, _BM, 2 * DEGW), lambda i: (0, i, 0))
_acc_spec = pl.BlockSpec((NC, _BM, 2 * DO), lambda i: (0, i, 0))
_row_spec = pl.BlockSpec((_BM, 2 * DO), lambda i: (i, 0))
_bias_spec = pl.BlockSpec((1, 2 * DO), lambda i: (0, 0))

_mm1 = pl.pallas_call(
    _mm1_body,
    grid=(NP // _BM,),
    in_specs=[_deg_spec,
              pl.BlockSpec((_BM, 2 * DF), lambda i: (i, 0)),
              pl.BlockSpec((2 * DF, 2 * DO), lambda i: (0, 0))],
    out_specs=[_row_spec, _row_spec],
    out_shape=[jax.ShapeDtypeStruct((NP, 2 * DO), jnp.float32),
               jax.ShapeDtypeStruct((NP, 2 * DO), jnp.float32)],
)

_mm2 = pl.pallas_call(
    _mm2_body,
    grid=(NP // _BM,),
    in_specs=[_row_spec, _acc_spec, _row_spec, _bias_spec,
              pl.BlockSpec((2 * DO, 2 * DO), lambda i: (0, 0))],
    out_specs=_row_spec,
    out_shape=jax.ShapeDtypeStruct((NP, 2 * DO), jnp.float32),
)

_mm3 = pl.pallas_call(
    _mm3_body,
    grid=(NP // _BM,),
    in_specs=[_row_spec, _acc_spec, _row_spec, _bias_spec],
    out_specs=_row_spec,
    out_shape=jax.ShapeDtypeStruct((NP, 2 * DO), jnp.float32),
)


def _blockdiag(W):
    k, m = W.shape
    Wbd = jnp.zeros((2 * k, 2 * m), W.dtype)
    return Wbd.at[:k, :m].set(W).at[k:, m:].set(W)


def kernel(x, edge_index, W0, b0, W1, b1):
    ei = edge_index.astype(jnp.int32)
    src_r = ei[0].reshape(NC, NS, NCHUNK, CH)
    dst_r = ei[1].reshape(NC, NS, NCHUNK, CH)
    xp = x.reshape(NP, 2 * DF)
    b0p = jnp.concatenate([b0, b0]).reshape(1, 2 * DO)
    b1p = jnp.concatenate([b1, b1]).reshape(1, 2 * DO)
    deg_parts = _deg_call(dst_r)
    degp = deg_parts.reshape(NC, NP, 2 * DEGW)
    g0p, dinvp = _mm1(degp, xp, _blockdiag(W0))
    g0 = g0p.reshape(N, DO)
    acc0 = _agg_call(g0, src_r, dst_r)
    g1p = _mm2(dinvp, acc0.reshape(NC, NP, 2 * DO), g0p, b0p, _blockdiag(W1))
    acc1 = _agg_call(g1p.reshape(N, DO), src_r, dst_r)
    outp = _mm3(dinvp, acc1.reshape(NC, NP, 2 * DO), g1p, b1p)
    return outp.reshape(N, DO)


# speedup vs baseline: 1.2677x; 1.0399x over previous
import functools

import jax
import jax.numpy as jnp
from jax import lax
from jax.experimental import pallas as pl
from jax.experimental.pallas import tpu as pltpu
from jax.experimental.pallas import tpu_sc as plsc

N = 10000
E = 320000
DF = 128
DO = 64
NC = 2
NS = 16
EPT = E // (NC * NS)
CH = 80
NCHUNK = EPT // CH
NBUF = 5
ABUF = 10
SLACK = 5
RPT = N // NS
DEGW = 16
NP = N // 2

_MESH = plsc.VectorSubcoreMesh(core_axis_name="c", subcore_axis_name="s")


def _copy_out(shared, out_hbm, c, s):
    pltpu.sync_copy(shared.at[pl.ds(s * 624, 624)],
                    out_hbm.at[c, pl.ds(s * 624, 624)])

    @pl.when(s == NS - 1)
    def _tail():
        pltpu.sync_copy(shared.at[pl.ds(9984, 16)],
                        out_hbm.at[c, pl.ds(9984, 16)])


def _deg_body(dst_hbm, out_hbm, idx_v, ones_v, zrow_v, deg_sh, sem, lsem):
    c = lax.axis_index("c")
    s = lax.axis_index("s")
    ldma = pltpu.async_copy(dst_hbm.at[c, s], idx_v, lsem)

    def fill(i, _):
        zrow_v[i, :] = jnp.zeros((DEGW,), jnp.float32)
        return _

    lax.fori_loop(0, 125, fill, None)

    def fill_o(i, _):
        ones_v[i, :] = jnp.ones((DEGW,), jnp.float32)
        return _

    lax.fori_loop(0, CH, fill_o, None)
    for t in range(5):
        pltpu.sync_copy(zrow_v, deg_sh.at[pl.ds(s * RPT + t * 125, 125)])
    ldma.wait()
    plsc.subcore_barrier()

    def fire(g):
        for b in range(NBUF):
            pltpu.async_copy(ones_v, deg_sh.at[idx_v.at[g * NBUF + b]], sem,
                             add=True)

    def drain(g):
        for b in range(NBUF):
            pltpu.make_async_copy(ones_v, deg_sh.at[idx_v.at[g * NBUF]],
                                  sem).wait()

    fire(0)

    def grp(g, _):
        fire(g)
        drain(g - 1)
        return _

    lax.fori_loop(1, NCHUNK // NBUF, grp, None)
    drain(NCHUNK // NBUF - 1)
    plsc.subcore_barrier()
    _copy_out(deg_sh, out_hbm, c, s)


_deg_call = functools.partial(
    pl.kernel,
    out_type=jax.ShapeDtypeStruct((NC, N, DEGW), jnp.float32),
    mesh=_MESH,
    scratch_types=[
        pltpu.VMEM((NCHUNK, CH), jnp.int32),
        pltpu.VMEM((CH, DEGW), jnp.float32),
        pltpu.VMEM((125, DEGW), jnp.float32),
        pltpu.VMEM_SHARED((N, DEGW), jnp.float32),
        pltpu.SemaphoreType.DMA,
        pltpu.SemaphoreType.DMA,
    ],
)(_deg_body)


def _agg_body(g_hbm, src_hbm, dst_hbm, out_hbm, srci, dsti, rbs, zb, acc_sh,
              gsems, lsems):
    c = lax.axis_index("c")
    s = lax.axis_index("s")
    ldma0 = pltpu.async_copy(src_hbm.at[c, s], srci, lsems[0])
    ldma1 = pltpu.async_copy(dst_hbm.at[c, s], dsti, lsems[1])

    def fz(i, _):
        for k in range(DO // 16):
            zb[i, pl.ds(k * 16, 16)] = jnp.zeros((16,), jnp.float32)
        return _

    lax.fori_loop(0, 125, fz, None)
    for t in range(5):
        pltpu.sync_copy(zb, acc_sh.at[pl.ds(s * RPT + t * 125, 125)])
    ldma0.wait()
    ldma1.wait()
    plsc.subcore_barrier()

    for b in range(NBUF):
        pltpu.async_copy(g_hbm.at[srci.at[b]], rbs[b], gsems[b])

    def grp(g, _):
        for b in range(NBUF):
            j = g * NBUF + b
            pltpu.make_async_copy(g_hbm.at[srci.at[j]], rbs[b],
                                  gsems[b]).wait()
            pltpu.sync_copy(rbs[b], acc_sh.at[dsti.at[j]], add=True)
            pltpu.async_copy(g_hbm.at[srci.at[j + NBUF]], rbs[b], gsems[b])
        return _

    lax.fori_loop(0, NCHUNK // NBUF - 1, grp, None)
    for b in range(NBUF):
        j = NCHUNK - NBUF + b
        pltpu.make_async_copy(g_hbm.at[srci.at[j]], rbs[b], gsems[b]).wait()
        pltpu.sync_copy(rbs[b], acc_sh.at[dsti.at[j]], add=True)
    plsc.subcore_barrier()
    _copy_out(acc_sh, out_hbm, c, s)


_agg_call = functools.partial(
    pl.kernel,
    out_type=jax.ShapeDtypeStruct((NC, N, DO), jnp.float32),
    mesh=_MESH,
    scratch_types=[
        pltpu.VMEM((NCHUNK, CH), jnp.int32),
        pltpu.VMEM((NCHUNK, CH), jnp.int32),
        [pltpu.VMEM((CH, DO), jnp.float32)] * NBUF,
        pltpu.VMEM((125, DO), jnp.float32),
        pltpu.VMEM_SHARED((N, DO), jnp.float32),
        [pltpu.SemaphoreType.DMA] * NBUF,
        [pltpu.SemaphoreType.DMA] * 2,
    ],
    compiler_params=pltpu.CompilerParams(use_tc_tiling_on_sc=False),
)(_agg_body)


_BM = 1000


def _dinvp_of(deg_ref):
    s = deg_ref[0] + deg_ref[1]
    de = lax.rsqrt(1.0 + jnp.sum(s[:, :DEGW], axis=1, keepdims=True))
    do = lax.rsqrt(1.0 + jnp.sum(s[:, DEGW:], axis=1, keepdims=True))
    return jnp.concatenate([jnp.broadcast_to(de, (_BM, DO)),
                            jnp.broadcast_to(do, (_BM, DO))], axis=1)


def _mm1_body(deg_ref, x_ref, w_ref, g_ref, dinvp_ref):
    dinvp = _dinvp_of(deg_ref)
    dinvp_ref[...] = dinvp
    g_ref[...] = jnp.dot(x_ref[...], w_ref[...],
                         preferred_element_type=jnp.float32) * dinvp


def _mm2_body(dinvp_ref, acc_ref, g0_ref, b0_ref, w1_ref, g1_ref):
    dinvp = dinvp_ref[...]
    t = (acc_ref[0] + acc_ref[1] + g0_ref[...]) * dinvp + b0_ref[...]
    t = jnp.maximum(t, 0.0)
    g1_ref[...] = jnp.dot(t, w1_ref[...],
                          preferred_element_type=jnp.float32) * dinvp


def _mm3_body(dinvp_ref, acc_ref, g1_ref, b1_ref, out_ref):
    out_ref[...] = ((acc_ref[0] + acc_ref[1] + g1_ref[...]) * dinvp_ref[...]
                    + b1_ref[...])


_deg_spec = pl.BlockSpec((NC, _BM, 2 * DEGW), lambda i: (0, i, 0))
_acc_spec = pl.BlockSpec((NC, _BM, 2 * DO), lambda i: (0, i, 0))
_row_spec = pl.BlockSpec((_BM, 2 * DO), lambda i: (i, 0))
_bias_spec = pl.BlockSpec((1, 2 * DO), lambda i: (0, 0))

_mm1 = pl.pallas_call(
    _mm1_body,
    grid=(NP // _BM,),
    in_specs=[_deg_spec,
              pl.BlockSpec((_BM, 2 * DF), lambda i: (i, 0)),
              pl.BlockSpec((2 * DF, 2 * DO), lambda i: (0, 0))],
    out_specs=[_row_spec, _row_spec],
    out_shape=[jax.ShapeDtypeStruct((NP, 2 * DO), jnp.float32),
               jax.ShapeDtypeStruct((NP, 2 * DO), jnp.float32)],
)

_mm2 = pl.pallas_call(
    _mm2_body,
    grid=(NP // _BM,),
    in_specs=[_row_spec, _acc_spec, _row_spec, _bias_spec,
              pl.BlockSpec((2 * DO, 2 * DO), lambda i: (0, 0))],
    out_specs=_row_spec,
    out_shape=jax.ShapeDtypeStruct((NP, 2 * DO), jnp.float32),
)

_mm3 = pl.pallas_call(
    _mm3_body,
    grid=(NP // _BM,),
    in_specs=[_row_spec, _acc_spec, _row_spec, _bias_spec],
    out_specs=_row_spec,
    out_shape=jax.ShapeDtypeStruct((NP, 2 * DO), jnp.float32),
)


def _blockdiag(W):
    k, m = W.shape
    Wbd = jnp.zeros((2 * k, 2 * m), W.dtype)
    return Wbd.at[:k, :m].set(W).at[k:, m:].set(W)


def kernel(x, edge_index, W0, b0, W1, b1):
    ei = edge_index.astype(jnp.int32)
    src_r = ei[0].reshape(NC, NS, NCHUNK, CH)
    dst_r = ei[1].reshape(NC, NS, NCHUNK, CH)
    xp = x.reshape(NP, 2 * DF)
    b0p = jnp.concatenate([b0, b0]).reshape(1, 2 * DO)
    b1p = jnp.concatenate([b1, b1]).reshape(1, 2 * DO)
    deg_parts = _deg_call(dst_r)
    degp = deg_parts.reshape(NC, NP, 2 * DEGW)
    g0p, dinvp = _mm1(degp, xp, _blockdiag(W0))
    g0 = g0p.reshape(N, DO)
    acc0 = _agg_call(g0, src_r, dst_r)
    g1p = _mm2(dinvp, acc0.reshape(NC, NP, 2 * DO), g0p, b0p, _blockdiag(W1))
    acc1 = _agg_call(g1p.reshape(N, DO), src_r, dst_r)
    outp = _mm3(dinvp, acc1.reshape(NC, NP, 2 * DO), g1p, b1p)
    return outp.reshape(N, DO)
